# baseline (speedup 1.0000x reference)
import functools

import jax
import jax.numpy as jnp
from jax import lax
from jax.experimental import pallas as pl
from jax.experimental.pallas import tpu as pltpu
from jax.experimental.pallas import tpu_sc as plsc

NC = 2
NS = 16
NW = NC * NS
_SC_PARAMS = pltpu.CompilerParams(needs_layout_passes=False)


def _tile_ids(TPB):
  cid = lax.axis_index("c")
  sid = lax.axis_index("s")
  wid = cid * NS + sid
  return wid, wid // TPB, wid % TPB


def _make_edges_deg(B, C, EPB):
  TPB = NW // B
  EPT = EPB // TPB
  ITERS = EPT // 16
  E2 = B * EPB
  SH = max((C - 1).bit_length(), 1)
  mesh = plsc.VectorSubcoreMesh(core_axis_name="c", subcore_axis_name="s")

  @functools.partial(
      pl.kernel,
      out_type=[
          jax.ShapeDtypeStruct((E2,), jnp.int32),
          jax.ShapeDtypeStruct((TPB * B * C,), jnp.float32),
      ],
      mesh=mesh,
      compiler_params=_SC_PARAMS,
      scratch_types=[
          pltpu.VMEM((EPT,), jnp.int32),
          pltpu.VMEM((EPT,), jnp.int32),
          pltpu.VMEM((C,), jnp.float32),
      ],
  )
  def edges_deg(src_hbm, dst_hbm, eloc_hbm, deg_hbm, src_v, dst_v, acc_v):
    wid, b, q = _tile_ids(TPB)
    e0 = wid * EPT
    pltpu.sync_copy(src_hbm.at[pl.ds(e0, EPT)], src_v)
    pltpu.sync_copy(dst_hbm.at[pl.ds(e0, EPT)], dst_v)
    offv = jnp.full((16,), b * C, jnp.int32)
    zv = jnp.zeros((16,), jnp.float32)
    ones = jnp.ones((16,), jnp.float32)

    @plsc.parallel_loop(0, C // 16, 1, unroll=4)
    def _(i):
      acc_v[pl.ds(i * 16, 16)] = zv

    @plsc.parallel_loop(0, ITERS, 1, unroll=4)
    def _(i):
      dv = dst_v[pl.ds(i * 16, 16)] - offv
      sv = src_v[pl.ds(i * 16, 16)] - offv
      plsc.addupdate_scatter(acc_v, [dv], ones)
      src_v[pl.ds(i * 16, 16)] = jnp.left_shift(dv, SH) + sv

    pltpu.sync_copy(src_v, eloc_hbm.at[pl.ds(e0, EPT)])
    pltpu.sync_copy(acc_v, deg_hbm.at[pl.ds(q * (B * C) + b * C, C)])

  return edges_deg


def _make_spmv(Ttot, t0, T, B, C, EPB, CG):
  TPB = NW // B
  EPT = EPB // TPB
  ITERS = EPT // 16
  SH = max((C - 1).bit_length(), 1)
  MASK = (1 << SH) - 1
  mesh = plsc.VectorSubcoreMesh(core_axis_name="c", subcore_axis_name="s")

  @functools.partial(
      pl.kernel,
      out_type=jax.ShapeDtypeStruct((T * TPB * B * C,), jnp.float32),
      mesh=mesh,
      compiler_params=_SC_PARAMS,
      scratch_types=(
          [pltpu.VMEM((EPT,), jnp.int32)]
          + [pltpu.VMEM((C,), jnp.float32) for _ in range(2 * CG)]
      ),
  )
  def spmv(y_hbm, eloc_hbm, zeros_hbm, out_hbm, idx_v, *tv):
    tabs, accs = tv[:CG], tv[CG:]
    wid, b, q = _tile_ids(TPB)
    e0 = wid * EPT
    pltpu.sync_copy(eloc_hbm.at[pl.ds(e0, EPT)], idx_v)
    maskv = jnp.full((16,), MASK, jnp.int32)

    for g0 in range(0, T, CG):
      for gi in range(CG):
        pltpu.sync_copy(
            y_hbm.at[pl.ds((t0 + g0 + gi) * (B * C) + b * C, C)], tabs[gi])
        pltpu.sync_copy(zeros_hbm, accs[gi])

      @plsc.parallel_loop(0, ITERS, 1, unroll=8)
      def _(i):
        ev = idx_v[pl.ds(i * 16, 16)]
        sv = jnp.bitwise_and(ev, maskv)
        dv = jnp.right_shift(ev, SH)
        for gi in range(CG):
          vals = plsc.load_gather(tabs[gi], [sv])
          plsc.addupdate_scatter(accs[gi], [dv], vals)

      for gi in range(CG):
        pltpu.sync_copy(
            accs[gi],
            out_hbm.at[pl.ds(((g0 + gi) * TPB + q) * (B * C) + b * C, C)])

  return spmv


def _pick_blk(n):
  for blk in (3200, 640, 1280, 512, 256, 128):
    if n % blk == 0:
      return blk
  return n


def _sig(x):
  return 0.5 * jnp.tanh(0.5 * x) + 0.5


def _lstm(gates, c, HID):
  ig = _sig(gates[0 * HID:1 * HID])
  fg = _sig(gates[1 * HID:2 * HID])
  gg = jnp.tanh(gates[2 * HID:3 * HID])
  og = _sig(gates[3 * HID:4 * HID])
  c_new = c * fg + ig * gg
  h_new = og * jnp.tanh(c_new)
  return h_new, c_new


def _prep_kernel(deg4_ref, pm25_ref, feat_ref, w1blk_ref, pblk_ref,
                 dinv_ref, ysc_ref):
  deg = jnp.sum(deg4_ref[...], axis=0)
  dinv = jnp.where(deg > 0, lax.rsqrt(jnp.maximum(deg, 1e-12)), 0.0)
  dinv_ref[...] = dinv[None, :]
  q_all = (jnp.dot(w1blk_ref[...], feat_ref[...],
                   preferred_element_type=jnp.float32)
           + jnp.dot(pblk_ref[...], pm25_ref[...],
                     preferred_element_type=jnp.float32))
  ysc_ref[...] = dinv[None, :] * q_all


def _xg(a, f9, zsum, dinv, w0t, bc, extra):
  pre = jnp.dot(w0t[:, 1:], f9, preferred_element_type=jnp.float32)
  x = w0t[:, 0:1] * a[None, :] + pre + bc - dinv[None, :] * zsum
  if extra is not None:
    x = x + extra
  return _sig(x)


def _hist_kernel(pm25_ref, feat_ref, z_ref, dinv_ref, w0t_ref, bc_ref,
                 wfull_ref, sel_ref, wo_ref, bo_ref,
                 h_ref, c_ref, xn_ref, axn_ref, *, HIST, HID, BLK, INF):
  dinv = dinv_ref[0]
  w0t = w0t_ref[...]
  bc = bc_ref[...]
  wfull = wfull_ref[...]
  zs = jnp.dot(sel_ref[...], z_ref[...],
               preferred_element_type=jnp.float32)
  ones_row = jnp.ones((1, BLK), jnp.float32)
  h = jnp.zeros((HID, BLK), jnp.float32)
  c = jnp.zeros((HID, BLK), jnp.float32)
  for s in range(HIST):
    a = pm25_ref[s]
    f9 = feat_ref[INF * s:INF * (s + 1)]
    xg = _xg(a, f9, zs[2 * s:2 * s + 2], dinv, w0t, bc, None)
    xx = jnp.concatenate([a[None, :], f9, xg, h, ones_row], axis=0)
    gates = jnp.dot(wfull, xx, preferred_element_type=jnp.float32)
    h, c = _lstm(gates, c, HID)
  wo = wo_ref[...]
  xn = jnp.dot(wo, h, preferred_element_type=jnp.float32) + bo_ref[0, 0]
  h_ref[...] = h
  c_ref[...] = c
  xn_ref[...] = xn
  axn_ref[...] = dinv[None, :] * xn


def _step_kernel(h_in_ref, c_in_ref, xn_in_ref, feat_ref, z_ref, s_ref,
                 dinv_ref, w0t_ref, w1t_ref, bc_ref, wfull_ref, sel2_ref,
                 wo_ref, bo_ref, h_ref, c_ref, xn_ref, axn_ref,
                 *, HID, BLK):
  dinv = dinv_ref[0]
  a = xn_in_ref[0]
  f9 = feat_ref[...]
  zsum = jnp.dot(sel2_ref[...], z_ref[...],
                 preferred_element_type=jnp.float32)
  sd = dinv * jnp.sum(s_ref[0], axis=0)
  extra = -w1t_ref[...][:, 0:1] * sd[None, :]
  xg = _xg(a, f9, zsum, dinv, w0t_ref[...], bc_ref[...], extra)
  ones_row = jnp.ones((1, BLK), jnp.float32)
  xx = jnp.concatenate([a[None, :], f9, xg, h_in_ref[...], ones_row], axis=0)
  gates = jnp.dot(wfull_ref[...], xx, preferred_element_type=jnp.float32)
  h, c = _lstm(gates, c_in_ref[...], HID)
  xn = jnp.dot(wo_ref[...], h, preferred_element_type=jnp.float32) + bo_ref[0, 0]
  h_ref[...] = h
  c_ref[...] = c
  xn_ref[...] = xn
  axn_ref[...] = dinv[None, :] * xn


def kernel(pm25_hist, feature, edge_index, W0, W1, bC, Wx, bx, Wh, bh, Wo, bo):
  B, HIST, C = pm25_hist.shape
  PRED = feature.shape[1] - HIST
  T = HIST + PRED
  N = B * C
  INF = feature.shape[3]
  HID = Wh.shape[1]
  E2 = edge_index.shape[1]
  EPB = E2 // B
  TPB = NW // B

  src = edge_index[0].astype(jnp.int32)
  dst = edge_index[1].astype(jnp.int32)

  eloc, deg4 = _make_edges_deg(B, C, EPB)(src, dst)
  deg4 = deg4.reshape(TPB, N)

  pm25T = pm25_hist.transpose(1, 0, 2).reshape(HIST, N)
  featT = feature.transpose(1, 3, 0, 2).reshape(T * INF, N)
  GCN = W0.shape[1]
  w0t = W0.T
  w1t = W1.T
  bc2 = bC.reshape(-1, 1)
  bxh = (bx + bh).reshape(-1, 1)
  bo2 = bo.reshape(1, 1)
  wfull = jnp.concatenate([Wx, Wh, bxh], axis=1)
  w1blk = jnp.kron(jnp.eye(T, dtype=jnp.float32), W1[1:].T)
  pblk = jnp.kron(jnp.eye(T, dtype=jnp.float32),
                  W1[0:1].T)[:, :HIST]
  sel = jnp.kron(jnp.eye(2 * HIST, dtype=jnp.float32),
                 jnp.ones((1, TPB), jnp.float32))
  sel2 = jnp.kron(jnp.eye(2, dtype=jnp.float32),
                  jnp.ones((1, TPB), jnp.float32))

  BLK = _pick_blk(N)
  grid = (N // BLK,)
  tc_params = pltpu.CompilerParams(dimension_semantics=("parallel",))
  fullw = lambda shape: pl.BlockSpec(shape, lambda j: (0,) * len(shape))

  dinv, ysc = pl.pallas_call(
      _prep_kernel,
      grid=grid,
      compiler_params=tc_params,
      in_specs=[
          pl.BlockSpec((TPB, BLK), lambda j: (0, j)),
          pl.BlockSpec((HIST, BLK), lambda j: (0, j)),
          pl.BlockSpec((T * INF, BLK), lambda j: (0, j)),
          fullw(w1blk.shape), fullw(pblk.shape),
      ],
      out_specs=[
          pl.BlockSpec((1, BLK), lambda j: (0, j)),
          pl.BlockSpec((2 * T, BLK), lambda j: (0, j)),
      ],
      out_shape=[
          jax.ShapeDtypeStruct((1, N), jnp.float32),
          jax.ShapeDtypeStruct((2 * T, N), jnp.float32),
      ],
  )(deg4, pm25T, featT, w1blk, pblk)

  yflat = ysc.reshape(-1)
  czeros = jnp.zeros((C,), jnp.float32)
  zh = _make_spmv(2 * T, 0, 2 * HIST, B, C, EPB, 4)(yflat, eloc, czeros)
  zh = zh.reshape(2 * HIST * TPB, N)
  zp = _make_spmv(2 * T, 2 * HIST, 2 * PRED, B, C, EPB, 4)(yflat, eloc, czeros)
  zp = zp.reshape(2 * PRED * TPB, N)

  h, c, xn, axn = pl.pallas_call(
      functools.partial(_hist_kernel, HIST=HIST, HID=HID, BLK=BLK, INF=INF),
      grid=grid,
      compiler_params=tc_params,
      in_specs=[
          pl.BlockSpec((HIST, BLK), lambda j: (0, j)),
          pl.BlockSpec((HIST * INF, BLK), lambda j: (0, j)),
          pl.BlockSpec((2 * HIST * TPB, BLK), lambda j: (0, j)),
          pl.BlockSpec((1, BLK), lambda j: (0, j)),
          fullw(w0t.shape), fullw(bc2.shape), fullw(wfull.shape),
          fullw(sel.shape), fullw(Wo.shape), fullw(bo2.shape),
      ],
      out_specs=[
          pl.BlockSpec((HID, BLK), lambda j: (0, j)),
          pl.BlockSpec((HID, BLK), lambda j: (0, j)),
          pl.BlockSpec((1, BLK), lambda j: (0, j)),
          pl.BlockSpec((1, BLK), lambda j: (0, j)),
      ],
      out_shape=[
          jax.ShapeDtypeStruct((HID, N), jnp.float32),
          jax.ShapeDtypeStruct((HID, N), jnp.float32),
          jax.ShapeDtypeStruct((1, N), jnp.float32),
          jax.ShapeDtypeStruct((1, N), jnp.float32),
      ],
  )(pm25T, featT, zh, dinv, w0t, bc2, wfull, sel, Wo, bo2)

  spmv1 = _make_spmv(1, 0, 1, B, C, EPB, 1)
  SBLK = 16000 if N % 16000 == 0 else BLK
  sgrid = (N // SBLK,)

  preds = []
  for i in range(PRED):
    sraw = spmv1(axn.reshape(-1), eloc, czeros).reshape(1, TPB, N)
    fi = i
    feat_i = lax.slice_in_dim(featT, (HIST + i) * INF, (HIST + i + 1) * INF,
                              axis=0)

    step = pl.pallas_call(
        functools.partial(_step_kernel, HID=HID, BLK=SBLK),
        grid=sgrid,
        compiler_params=tc_params,
        in_specs=[
            pl.BlockSpec((HID, SBLK), lambda j: (0, j)),
            pl.BlockSpec((HID, SBLK), lambda j: (0, j)),
            pl.BlockSpec((1, SBLK), lambda j: (0, j)),
            pl.BlockSpec((INF, SBLK), lambda j: (0, j)),
            pl.BlockSpec((2 * TPB, SBLK), lambda j, fi=fi: (fi, j)),
            pl.BlockSpec((1, TPB, SBLK), lambda j: (0, 0, j)),
            pl.BlockSpec((1, SBLK), lambda j: (0, j)),
            fullw(w0t.shape), fullw(w1t.shape), fullw(bc2.shape),
            fullw(wfull.shape), fullw(sel2.shape),
            fullw(Wo.shape), fullw(bo2.shape),
        ],
        out_specs=[
            pl.BlockSpec((HID, SBLK), lambda j: (0, j)),
            pl.BlockSpec((HID, SBLK), lambda j: (0, j)),
            pl.BlockSpec((1, SBLK), lambda j: (0, j)),
            pl.BlockSpec((1, SBLK), lambda j: (0, j)),
        ],
        out_shape=[
            jax.ShapeDtypeStruct((HID, N), jnp.float32),
            jax.ShapeDtypeStruct((HID, N), jnp.float32),
            jax.ShapeDtypeStruct((1, N), jnp.float32),
            jax.ShapeDtypeStruct((1, N), jnp.float32),
        ],
    )
    h, c, xn, axn = step(h, c, xn, feat_i, zp, sraw, dinv,
                         w0t, w1t, bc2, wfull, sel2, Wo, bo2)
    preds.append(xn)
  out = jnp.concatenate(preds, axis=0).reshape(PRED, B, C).transpose(1, 0, 2)
  return out


# speedup vs baseline: 1.0201x; 1.0201x over previous
import functools

import jax
import jax.numpy as jnp
from jax import lax
from jax.experimental import pallas as pl
from jax.experimental.pallas import tpu as pltpu
from jax.experimental.pallas import tpu_sc as plsc

NC = 2
NS = 16
NW = NC * NS
_SC_PARAMS = pltpu.CompilerParams(needs_layout_passes=False)


def _tile_ids(TPB):
  cid = lax.axis_index("c")
  sid = lax.axis_index("s")
  wid = cid * NS + sid
  return wid, wid // TPB, wid % TPB


def _make_edges_deg(B, C, EPB):
  TPB = NW // B
  EPT = EPB // TPB
  ITERS = EPT // 16
  E2 = B * EPB
  SH = max((C - 1).bit_length(), 1)
  mesh = plsc.VectorSubcoreMesh(core_axis_name="c", subcore_axis_name="s")

  @functools.partial(
      pl.kernel,
      out_type=[
          jax.ShapeDtypeStruct((E2,), jnp.int32),
          jax.ShapeDtypeStruct((TPB * B * C,), jnp.float32),
      ],
      mesh=mesh,
      compiler_params=_SC_PARAMS,
      scratch_types=[
          pltpu.VMEM((EPT,), jnp.int32),
          pltpu.VMEM((EPT,), jnp.int32),
          pltpu.VMEM((C,), jnp.float32),
      ],
  )
  def edges_deg(src_hbm, dst_hbm, eloc_hbm, deg_hbm, src_v, dst_v, acc_v):
    wid, b, q = _tile_ids(TPB)
    e0 = wid * EPT
    pltpu.sync_copy(src_hbm.at[pl.ds(e0, EPT)], src_v)
    pltpu.sync_copy(dst_hbm.at[pl.ds(e0, EPT)], dst_v)
    offv = jnp.full((16,), b * C, jnp.int32)
    zv = jnp.zeros((16,), jnp.float32)
    ones = jnp.ones((16,), jnp.float32)

    @plsc.parallel_loop(0, C // 16, 1, unroll=4)
    def _(i):
      acc_v[pl.ds(i * 16, 16)] = zv

    @plsc.parallel_loop(0, ITERS, 1, unroll=4)
    def _(i):
      dv = dst_v[pl.ds(i * 16, 16)] - offv
      sv = src_v[pl.ds(i * 16, 16)] - offv
      plsc.addupdate_scatter(acc_v, [dv], ones)
      src_v[pl.ds(i * 16, 16)] = jnp.left_shift(dv, SH) + sv

    pltpu.sync_copy(src_v, eloc_hbm.at[pl.ds(e0, EPT)])
    pltpu.sync_copy(acc_v, deg_hbm.at[pl.ds(q * (B * C) + b * C, C)])

  return edges_deg


def _make_spmv(Ttot, t0, T, B, C, EPB, CG):
  TPB = NW // B
  EPT = EPB // TPB
  ITERS = EPT // 16
  SH = max((C - 1).bit_length(), 1)
  MASK = (1 << SH) - 1
  mesh = plsc.VectorSubcoreMesh(core_axis_name="c", subcore_axis_name="s")

  @functools.partial(
      pl.kernel,
      out_type=jax.ShapeDtypeStruct((T * TPB * B * C,), jnp.float32),
      mesh=mesh,
      compiler_params=_SC_PARAMS,
      scratch_types=(
          [pltpu.VMEM((EPT,), jnp.int32)]
          + [pltpu.VMEM((C,), jnp.float32) for _ in range(2 * CG)]
          + [pltpu.SemaphoreType.DMA]
      ),
  )
  def spmv(y_hbm, eloc_hbm, zeros_hbm, out_hbm, idx_v, *tv):
    tabs, accs = tv[:CG], tv[CG:2 * CG]
    sem = tv[2 * CG]
    wid, b, q = _tile_ids(TPB)
    e0 = wid * EPT
    pltpu.sync_copy(eloc_hbm.at[pl.ds(e0, EPT)], idx_v)
    maskv = jnp.full((16,), MASK, jnp.int32)

    for g0 in range(0, T, CG):
      cps = []
      for gi in range(CG):
        cps.append(pltpu.async_copy(
            y_hbm.at[pl.ds((t0 + g0 + gi) * (B * C) + b * C, C)],
            tabs[gi], sem))
        cps.append(pltpu.async_copy(zeros_hbm, accs[gi], sem))
      for cp in cps:
        cp.wait()

      @plsc.parallel_loop(0, ITERS, 1, unroll=4)
      def _(i):
        ev = idx_v[pl.ds(i * 16, 16)]
        sv = jnp.bitwise_and(ev, maskv)
        dv = jnp.right_shift(ev, SH)
        for gi in range(CG):
          vals = plsc.load_gather(tabs[gi], [sv])
          plsc.addupdate_scatter(accs[gi], [dv], vals)

      for gi in range(CG):
        pltpu.sync_copy(
            accs[gi],
            out_hbm.at[pl.ds(((g0 + gi) * TPB + q) * (B * C) + b * C, C)])

  return spmv


def _pick_blk(n):
  for blk in (3200, 640, 1280, 512, 256, 128):
    if n % blk == 0:
      return blk
  return n


def _sig(x):
  return 0.5 * jnp.tanh(0.5 * x) + 0.5


def _lstm(gates, c, HID):
  ig = _sig(gates[0 * HID:1 * HID])
  fg = _sig(gates[1 * HID:2 * HID])
  gg = jnp.tanh(gates[2 * HID:3 * HID])
  og = _sig(gates[3 * HID:4 * HID])
  c_new = c * fg + ig * gg
  h_new = og * jnp.tanh(c_new)
  return h_new, c_new


def _prep_kernel(deg4_ref, pm25_ref, feat_ref, w1blk_ref, pblk_ref,
                 dinv_ref, ysc_ref):
  deg = jnp.sum(deg4_ref[...], axis=0)
  dinv = jnp.where(deg > 0, lax.rsqrt(jnp.maximum(deg, 1e-12)), 0.0)
  dinv_ref[...] = dinv[None, :]
  q_all = (jnp.dot(w1blk_ref[...], feat_ref[...],
                   preferred_element_type=jnp.float32)
           + jnp.dot(pblk_ref[...], pm25_ref[...],
                     preferred_element_type=jnp.float32))
  ysc_ref[...] = dinv[None, :] * q_all


def _xg(a, f9, zsum, dinv, w0t, bc, extra):
  pre = jnp.dot(w0t[:, 1:], f9, preferred_element_type=jnp.float32)
  x = w0t[:, 0:1] * a[None, :] + pre + bc - dinv[None, :] * zsum
  if extra is not None:
    x = x + extra
  return _sig(x)


def _hist_kernel(pm25_ref, feat_ref, z_ref, dinv_ref, w0t_ref, bc_ref,
                 wfull_ref, sel_ref, wo_ref, bo_ref,
                 h_ref, c_ref, xn_ref, axn_ref, *, HIST, HID, BLK, INF):
  dinv = dinv_ref[0]
  w0t = w0t_ref[...]
  bc = bc_ref[...]
  wfull = wfull_ref[...]
  zs = jnp.dot(sel_ref[...], z_ref[...],
               preferred_element_type=jnp.float32)
  ones_row = jnp.ones((1, BLK), jnp.float32)
  h = jnp.zeros((HID, BLK), jnp.float32)
  c = jnp.zeros((HID, BLK), jnp.float32)
  for s in range(HIST):
    a = pm25_ref[s]
    f9 = feat_ref[INF * s:INF * (s + 1)]
    xg = _xg(a, f9, zs[2 * s:2 * s + 2], dinv, w0t, bc, None)
    xx = jnp.concatenate([a[None, :], f9, xg, h, ones_row], axis=0)
    gates = jnp.dot(wfull, xx, preferred_element_type=jnp.float32)
    h, c = _lstm(gates, c, HID)
  wo = wo_ref[...]
  xn = jnp.dot(wo, h, preferred_element_type=jnp.float32) + bo_ref[0, 0]
  h_ref[...] = h
  c_ref[...] = c
  xn_ref[...] = xn
  axn_ref[...] = dinv[None, :] * xn


def _step_kernel(h_in_ref, c_in_ref, xn_in_ref, feat_ref, z_ref, s_ref,
                 dinv_ref, w0t_ref, w1t_ref, bc_ref, wfull_ref, sel2_ref,
                 wo_ref, bo_ref, h_ref, c_ref, xn_ref, axn_ref,
                 *, HID, BLK):
  dinv = dinv_ref[0]
  a = xn_in_ref[0]
  f9 = feat_ref[...]
  zsum = jnp.dot(sel2_ref[...], z_ref[...],
                 preferred_element_type=jnp.float32)
  sd = dinv * jnp.sum(s_ref[0], axis=0)
  extra = -w1t_ref[...][:, 0:1] * sd[None, :]
  xg = _xg(a, f9, zsum, dinv, w0t_ref[...], bc_ref[...], extra)
  ones_row = jnp.ones((1, BLK), jnp.float32)
  xx = jnp.concatenate([a[None, :], f9, xg, h_in_ref[...], ones_row], axis=0)
  gates = jnp.dot(wfull_ref[...], xx, preferred_element_type=jnp.float32)
  h, c = _lstm(gates, c_in_ref[...], HID)
  xn = jnp.dot(wo_ref[...], h, preferred_element_type=jnp.float32) + bo_ref[0, 0]
  h_ref[...] = h
  c_ref[...] = c
  xn_ref[...] = xn
  axn_ref[...] = dinv[None, :] * xn


def kernel(pm25_hist, feature, edge_index, W0, W1, bC, Wx, bx, Wh, bh, Wo, bo):
  B, HIST, C = pm25_hist.shape
  PRED = feature.shape[1] - HIST
  T = HIST + PRED
  N = B * C
  INF = feature.shape[3]
  HID = Wh.shape[1]
  E2 = edge_index.shape[1]
  EPB = E2 // B
  TPB = NW // B

  src = edge_index[0].astype(jnp.int32)
  dst = edge_index[1].astype(jnp.int32)

  eloc, deg4 = _make_edges_deg(B, C, EPB)(src, dst)
  deg4 = deg4.reshape(TPB, N)

  pm25T = pm25_hist.transpose(1, 0, 2).reshape(HIST, N)
  featT = feature.transpose(1, 3, 0, 2).reshape(T * INF, N)
  GCN = W0.shape[1]
  w0t = W0.T
  w1t = W1.T
  bc2 = bC.reshape(-1, 1)
  bxh = (bx + bh).reshape(-1, 1)
  bo2 = bo.reshape(1, 1)
  wfull = jnp.concatenate([Wx, Wh, bxh], axis=1)
  w1blk = jnp.kron(jnp.eye(T, dtype=jnp.float32), W1[1:].T)
  pblk = jnp.kron(jnp.eye(T, dtype=jnp.float32),
                  W1[0:1].T)[:, :HIST]
  sel = jnp.kron(jnp.eye(2 * HIST, dtype=jnp.float32),
                 jnp.ones((1, TPB), jnp.float32))
  sel2 = jnp.kron(jnp.eye(2, dtype=jnp.float32),
                  jnp.ones((1, TPB), jnp.float32))

  BLK = _pick_blk(N)
  grid = (N // BLK,)
  tc_params = pltpu.CompilerParams(dimension_semantics=("parallel",))
  fullw = lambda shape: pl.BlockSpec(shape, lambda j: (0,) * len(shape))

  dinv, ysc = pl.pallas_call(
      _prep_kernel,
      grid=grid,
      compiler_params=tc_params,
      in_specs=[
          pl.BlockSpec((TPB, BLK), lambda j: (0, j)),
          pl.BlockSpec((HIST, BLK), lambda j: (0, j)),
          pl.BlockSpec((T * INF, BLK), lambda j: (0, j)),
          fullw(w1blk.shape), fullw(pblk.shape),
      ],
      out_specs=[
          pl.BlockSpec((1, BLK), lambda j: (0, j)),
          pl.BlockSpec((2 * T, BLK), lambda j: (0, j)),
      ],
      out_shape=[
          jax.ShapeDtypeStruct((1, N), jnp.float32),
          jax.ShapeDtypeStruct((2 * T, N), jnp.float32),
      ],
  )(deg4, pm25T, featT, w1blk, pblk)

  yflat = ysc.reshape(-1)
  czeros = jnp.zeros((C,), jnp.float32)
  zh = _make_spmv(2 * T, 0, 2 * HIST, B, C, EPB, 4)(yflat, eloc, czeros)
  zh = zh.reshape(2 * HIST * TPB, N)
  zp = _make_spmv(2 * T, 2 * HIST, 2 * PRED, B, C, EPB, 4)(yflat, eloc, czeros)
  zp = zp.reshape(2 * PRED * TPB, N)

  h, c, xn, axn = pl.pallas_call(
      functools.partial(_hist_kernel, HIST=HIST, HID=HID, BLK=BLK, INF=INF),
      grid=grid,
      compiler_params=tc_params,
      in_specs=[
          pl.BlockSpec((HIST, BLK), lambda j: (0, j)),
          pl.BlockSpec((HIST * INF, BLK), lambda j: (0, j)),
          pl.BlockSpec((2 * HIST * TPB, BLK), lambda j: (0, j)),
          pl.BlockSpec((1, BLK), lambda j: (0, j)),
          fullw(w0t.shape), fullw(bc2.shape), fullw(wfull.shape),
          fullw(sel.shape), fullw(Wo.shape), fullw(bo2.shape),
      ],
      out_specs=[
          pl.BlockSpec((HID, BLK), lambda j: (0, j)),
          pl.BlockSpec((HID, BLK), lambda j: (0, j)),
          pl.BlockSpec((1, BLK), lambda j: (0, j)),
          pl.BlockSpec((1, BLK), lambda j: (0, j)),
      ],
      out_shape=[
          jax.ShapeDtypeStruct((HID, N), jnp.float32),
          jax.ShapeDtypeStruct((HID, N), jnp.float32),
          jax.ShapeDtypeStruct((1, N), jnp.float32),
          jax.ShapeDtypeStruct((1, N), jnp.float32),
      ],
  )(pm25T, featT, zh, dinv, w0t, bc2, wfull, sel, Wo, bo2)

  spmv1 = _make_spmv(1, 0, 1, B, C, EPB, 1)
  SBLK = 16000 if N % 16000 == 0 else BLK
  sgrid = (N // SBLK,)

  preds = []
  for i in range(PRED):
    sraw = spmv1(axn.reshape(-1), eloc, czeros).reshape(1, TPB, N)
    fi = i
    feat_i = lax.slice_in_dim(featT, (HIST + i) * INF, (HIST + i + 1) * INF,
                              axis=0)

    step = pl.pallas_call(
        functools.partial(_step_kernel, HID=HID, BLK=SBLK),
        grid=sgrid,
        compiler_params=tc_params,
        in_specs=[
            pl.BlockSpec((HID, SBLK), lambda j: (0, j)),
            pl.BlockSpec((HID, SBLK), lambda j: (0, j)),
            pl.BlockSpec((1, SBLK), lambda j: (0, j)),
            pl.BlockSpec((INF, SBLK), lambda j: (0, j)),
            pl.BlockSpec((2 * TPB, SBLK), lambda j, fi=fi: (fi, j)),
            pl.BlockSpec((1, TPB, SBLK), lambda j: (0, 0, j)),
            pl.BlockSpec((1, SBLK), lambda j: (0, j)),
            fullw(w0t.shape), fullw(w1t.shape), fullw(bc2.shape),
            fullw(wfull.shape), fullw(sel2.shape),
            fullw(Wo.shape), fullw(bo2.shape),
        ],
        out_specs=[
            pl.BlockSpec((HID, SBLK), lambda j: (0, j)),
            pl.BlockSpec((HID, SBLK), lambda j: (0, j)),
            pl.BlockSpec((1, SBLK), lambda j: (0, j)),
            pl.BlockSpec((1, SBLK), lambda j: (0, j)),
        ],
        out_shape=[
            jax.ShapeDtypeStruct((HID, N), jnp.float32),
            jax.ShapeDtypeStruct((HID, N), jnp.float32),
            jax.ShapeDtypeStruct((1, N), jnp.float32),
            jax.ShapeDtypeStruct((1, N), jnp.float32),
        ],
    )
    h, c, xn, axn = step(h, c, xn, feat_i, zp, sraw, dinv,
                         w0t, w1t, bc2, wfull, sel2, Wo, bo2)
    preds.append(xn)
  out = jnp.concatenate(preds, axis=0).reshape(PRED, B, C).transpose(1, 0, 2)
  return out


# speedup vs baseline: 1.0718x; 1.0507x over previous
import functools

import jax
import jax.numpy as jnp
from jax import lax
from jax.experimental import pallas as pl
from jax.experimental.pallas import tpu as pltpu
from jax.experimental.pallas import tpu_sc as plsc

NC = 2
NS = 16
NW = NC * NS
_SC_PARAMS = pltpu.CompilerParams(needs_layout_passes=False)


def _tile_ids(TPB):
  cid = lax.axis_index("c")
  sid = lax.axis_index("s")
  wid = cid * NS + sid
  return wid, wid // TPB, wid % TPB


def _make_edges_deg(B, C, EPB):
  TPB = NW // B
  EPT = EPB // TPB
  ITERS = EPT // 16
  E2 = B * EPB
  SH = max((C - 1).bit_length(), 1)
  mesh = plsc.VectorSubcoreMesh(core_axis_name="c", subcore_axis_name="s")

  @functools.partial(
      pl.kernel,
      out_type=[
          jax.ShapeDtypeStruct((E2,), jnp.int32),
          jax.ShapeDtypeStruct((TPB * B * C,), jnp.float32),
      ],
      mesh=mesh,
      compiler_params=_SC_PARAMS,
      scratch_types=[
          pltpu.VMEM((EPT,), jnp.int32),
          pltpu.VMEM((EPT,), jnp.int32),
          pltpu.VMEM((C,), jnp.float32),
      ],
  )
  def edges_deg(src_hbm, dst_hbm, eloc_hbm, deg_hbm, src_v, dst_v, acc_v):
    wid, b, q = _tile_ids(TPB)
    e0 = wid * EPT
    pltpu.sync_copy(src_hbm.at[pl.ds(e0, EPT)], src_v)
    pltpu.sync_copy(dst_hbm.at[pl.ds(e0, EPT)], dst_v)
    offv = jnp.full((16,), b * C, jnp.int32)
    zv = jnp.zeros((16,), jnp.float32)
    ones = jnp.ones((16,), jnp.float32)

    @plsc.parallel_loop(0, C // 16, 1, unroll=4)
    def _(i):
      acc_v[pl.ds(i * 16, 16)] = zv

    @plsc.parallel_loop(0, ITERS, 1, unroll=4)
    def _(i):
      dv = dst_v[pl.ds(i * 16, 16)] - offv
      sv = src_v[pl.ds(i * 16, 16)] - offv
      plsc.addupdate_scatter(acc_v, [dv], ones)
      src_v[pl.ds(i * 16, 16)] = jnp.left_shift(dv, SH) + sv

    pltpu.sync_copy(src_v, eloc_hbm.at[pl.ds(e0, EPT)])
    pltpu.sync_copy(acc_v, deg_hbm.at[pl.ds(q * (B * C) + b * C, C)])

  return edges_deg


def _make_spmv(Ttot, t0, T, B, C, EPB, CG):
  TPB = NW // B
  EPT = EPB // TPB
  ITERS = EPT // 16
  SH = max((C - 1).bit_length(), 1)
  MASK = (1 << SH) - 1
  mesh = plsc.VectorSubcoreMesh(core_axis_name="c", subcore_axis_name="s")

  @functools.partial(
      pl.kernel,
      out_type=jax.ShapeDtypeStruct((T * TPB * B * C,), jnp.float32),
      mesh=mesh,
      compiler_params=_SC_PARAMS,
      scratch_types=(
          [pltpu.VMEM((EPT,), jnp.int32)]
          + [pltpu.VMEM((C,), jnp.float32) for _ in range(2 * CG)]
      ),
  )
  def spmv(y_hbm, eloc_hbm, out_hbm, idx_v, *tv):
    tabs, accs = tv[:CG], tv[CG:]
    wid, b, q = _tile_ids(TPB)
    e0 = wid * EPT
    pltpu.sync_copy(eloc_hbm.at[pl.ds(e0, EPT)], idx_v)
    zv = jnp.zeros((16,), jnp.float32)
    maskv = jnp.full((16,), MASK, jnp.int32)

    for g0 in range(0, T, CG):
      for gi in range(CG):
        pltpu.sync_copy(
            y_hbm.at[pl.ds((t0 + g0 + gi) * (B * C) + b * C, C)], tabs[gi])

      @plsc.parallel_loop(0, C // 16, 1, unroll=4)
      def _(i):
        for gi in range(CG):
          accs[gi][pl.ds(i * 16, 16)] = zv

      @plsc.parallel_loop(0, ITERS, 1, unroll=4)
      def _(i):
        ev = idx_v[pl.ds(i * 16, 16)]
        sv = jnp.bitwise_and(ev, maskv)
        dv = jnp.right_shift(ev, SH)
        for gi in range(CG):
          vals = plsc.load_gather(tabs[gi], [sv])
          plsc.addupdate_scatter(accs[gi], [dv], vals)

      for gi in range(CG):
        pltpu.sync_copy(
            accs[gi],
            out_hbm.at[pl.ds(((g0 + gi) * TPB + q) * (B * C) + b * C, C)])

  return spmv


def _pick_blk(n):
  for blk in (3200, 640, 1280, 512, 256, 128):
    if n % blk == 0:
      return blk
  return n


def _sig(x):
  return 0.5 * jnp.tanh(0.5 * x) + 0.5


def _lstm(gates, c, HID):
  ig = _sig(gates[0 * HID:1 * HID])
  fg = _sig(gates[1 * HID:2 * HID])
  gg = jnp.tanh(gates[2 * HID:3 * HID])
  og = _sig(gates[3 * HID:4 * HID])
  c_new = c * fg + ig * gg
  h_new = og * jnp.tanh(c_new)
  return h_new, c_new


def _prep_kernel(deg4_ref, pm25_ref, feat_ref, w1blk_ref, pblk_ref,
                 dinv_ref, ysc_ref):
  deg = jnp.sum(deg4_ref[...], axis=0)
  dinv = jnp.where(deg > 0, lax.rsqrt(jnp.maximum(deg, 1e-12)), 0.0)
  dinv_ref[...] = dinv[None, :]
  q_all = (jnp.dot(w1blk_ref[...], feat_ref[...],
                   preferred_element_type=jnp.float32)
           + jnp.dot(pblk_ref[...], pm25_ref[...],
                     preferred_element_type=jnp.float32))
  ysc_ref[...] = dinv[None, :] * q_all


def _xg(a, f9, zsum, dinv, w0t, bc, extra):
  pre = jnp.dot(w0t[:, 1:], f9, preferred_element_type=jnp.float32)
  x = w0t[:, 0:1] * a[None, :] + pre + bc - dinv[None, :] * zsum
  if extra is not None:
    x = x + extra
  return _sig(x)


def _hist_kernel(pm25_ref, feat_ref, z_ref, dinv_ref, w0t_ref, bc_ref,
                 wfull_ref, sel_ref, wo_ref, bo_ref,
                 h_ref, c_ref, xn_ref, axn_ref, *, HIST, HID, BLK, INF):
  dinv = dinv_ref[0]
  w0t = w0t_ref[...]
  bc = bc_ref[...]
  wfull = wfull_ref[...]
  zs = jnp.dot(sel_ref[...], z_ref[...],
               preferred_element_type=jnp.float32)
  ones_row = jnp.ones((1, BLK), jnp.float32)
  h = jnp.zeros((HID, BLK), jnp.float32)
  c = jnp.zeros((HID, BLK), jnp.float32)
  for s in range(HIST):
    a = pm25_ref[s]
    f9 = feat_ref[INF * s:INF * (s + 1)]
    xg = _xg(a, f9, zs[2 * s:2 * s + 2], dinv, w0t, bc, None)
    xx = jnp.concatenate([a[None, :], f9, xg, h, ones_row], axis=0)
    gates = jnp.dot(wfull, xx, preferred_element_type=jnp.float32)
    h, c = _lstm(gates, c, HID)
  wo = wo_ref[...]
  xn = jnp.dot(wo, h, preferred_element_type=jnp.float32) + bo_ref[0, 0]
  h_ref[...] = h
  c_ref[...] = c
  xn_ref[...] = xn
  axn_ref[...] = dinv[None, :] * xn


def _step_kernel(h_in_ref, c_in_ref, xn_in_ref, feat_ref, z_ref, s_ref,
                 dinv_ref, w0t_ref, w1t_ref, bc_ref, wfull_ref, sel2_ref,
                 wo_ref, bo_ref, h_ref, c_ref, xn_ref, axn_ref,
                 *, HID, BLK):
  dinv = dinv_ref[0]
  a = xn_in_ref[0]
  f9 = feat_ref[...]
  zsum = jnp.dot(sel2_ref[...], z_ref[...],
                 preferred_element_type=jnp.float32)
  sd = dinv * jnp.sum(s_ref[0], axis=0)
  extra = -w1t_ref[...][:, 0:1] * sd[None, :]
  xg = _xg(a, f9, zsum, dinv, w0t_ref[...], bc_ref[...], extra)
  ones_row = jnp.ones((1, BLK), jnp.float32)
  xx = jnp.concatenate([a[None, :], f9, xg, h_in_ref[...], ones_row], axis=0)
  gates = jnp.dot(wfull_ref[...], xx, preferred_element_type=jnp.float32)
  h, c = _lstm(gates, c_in_ref[...], HID)
  xn = jnp.dot(wo_ref[...], h, preferred_element_type=jnp.float32) + bo_ref[0, 0]
  h_ref[...] = h
  c_ref[...] = c
  xn_ref[...] = xn
  axn_ref[...] = dinv[None, :] * xn


def kernel(pm25_hist, feature, edge_index, W0, W1, bC, Wx, bx, Wh, bh, Wo, bo):
  B, HIST, C = pm25_hist.shape
  PRED = feature.shape[1] - HIST
  T = HIST + PRED
  N = B * C
  INF = feature.shape[3]
  HID = Wh.shape[1]
  E2 = edge_index.shape[1]
  EPB = E2 // B
  TPB = NW // B

  src = edge_index[0].astype(jnp.int32)
  dst = edge_index[1].astype(jnp.int32)

  eloc, deg4 = _make_edges_deg(B, C, EPB)(src, dst)
  deg4 = deg4.reshape(TPB, N)

  pm25T = pm25_hist.transpose(1, 0, 2).reshape(HIST, N)
  featT = feature.transpose(1, 3, 0, 2).reshape(T * INF, N)
  GCN = W0.shape[1]
  w0t = W0.T
  w1t = W1.T
  bc2 = bC.reshape(-1, 1)
  bxh = (bx + bh).reshape(-1, 1)
  bo2 = bo.reshape(1, 1)
  wfull = jnp.concatenate([Wx, Wh, bxh], axis=1)
  w1blk = jnp.kron(jnp.eye(T, dtype=jnp.float32), W1[1:].T)
  pblk = jnp.kron(jnp.eye(T, dtype=jnp.float32),
                  W1[0:1].T)[:, :HIST]
  sel = jnp.kron(jnp.eye(2 * HIST, dtype=jnp.float32),
                 jnp.ones((1, TPB), jnp.float32))
  sel2 = jnp.kron(jnp.eye(2, dtype=jnp.float32),
                  jnp.ones((1, TPB), jnp.float32))

  BLK = _pick_blk(N)
  grid = (N // BLK,)
  tc_params = pltpu.CompilerParams(dimension_semantics=("parallel",))
  fullw = lambda shape: pl.BlockSpec(shape, lambda j: (0,) * len(shape))

  dinv, ysc = pl.pallas_call(
      _prep_kernel,
      grid=grid,
      compiler_params=tc_params,
      in_specs=[
          pl.BlockSpec((TPB, BLK), lambda j: (0, j)),
          pl.BlockSpec((HIST, BLK), lambda j: (0, j)),
          pl.BlockSpec((T * INF, BLK), lambda j: (0, j)),
          fullw(w1blk.shape), fullw(pblk.shape),
      ],
      out_specs=[
          pl.BlockSpec((1, BLK), lambda j: (0, j)),
          pl.BlockSpec((2 * T, BLK), lambda j: (0, j)),
      ],
      out_shape=[
          jax.ShapeDtypeStruct((1, N), jnp.float32),
          jax.ShapeDtypeStruct((2 * T, N), jnp.float32),
      ],
  )(deg4, pm25T, featT, w1blk, pblk)

  yflat = ysc.reshape(-1)
  zh = _make_spmv(2 * T, 0, 2 * HIST, B, C, EPB, 4)(yflat, eloc)
  zh = zh.reshape(2 * HIST * TPB, N)
  zp = _make_spmv(2 * T, 2 * HIST, 2 * PRED, B, C, EPB, 4)(yflat, eloc)
  zp = zp.reshape(2 * PRED * TPB, N)

  h, c, xn, axn = pl.pallas_call(
      functools.partial(_hist_kernel, HIST=HIST, HID=HID, BLK=BLK, INF=INF),
      grid=grid,
      compiler_params=tc_params,
      in_specs=[
          pl.BlockSpec((HIST, BLK), lambda j: (0, j)),
          pl.BlockSpec((HIST * INF, BLK), lambda j: (0, j)),
          pl.BlockSpec((2 * HIST * TPB, BLK), lambda j: (0, j)),
          pl.BlockSpec((1, BLK), lambda j: (0, j)),
          fullw(w0t.shape), fullw(bc2.shape), fullw(wfull.shape),
          fullw(sel.shape), fullw(Wo.shape), fullw(bo2.shape),
      ],
      out_specs=[
          pl.BlockSpec((HID, BLK), lambda j: (0, j)),
          pl.BlockSpec((HID, BLK), lambda j: (0, j)),
          pl.BlockSpec((1, BLK), lambda j: (0, j)),
          pl.BlockSpec((1, BLK), lambda j: (0, j)),
      ],
      out_shape=[
          jax.ShapeDtypeStruct((HID, N), jnp.float32),
          jax.ShapeDtypeStruct((HID, N), jnp.float32),
          jax.ShapeDtypeStruct((1, N), jnp.float32),
          jax.ShapeDtypeStruct((1, N), jnp.float32),
      ],
  )(pm25T, featT, zh, dinv, w0t, bc2, wfull, sel, Wo, bo2)

  spmv1 = _make_spmv(1, 0, 1, B, C, EPB, 1)
  SBLK = 16000 if N % 16000 == 0 else BLK
  sgrid = (N // SBLK,)

  preds = []
  for i in range(PRED):
    sraw = spmv1(axn.reshape(-1), eloc).reshape(1, TPB, N)
    fi = i
    feat_i = lax.slice_in_dim(featT, (HIST + i) * INF, (HIST + i + 1) * INF,
                              axis=0)

    step = pl.pallas_call(
        functools.partial(_step_kernel, HID=HID, BLK=SBLK),
        grid=sgrid,
        compiler_params=tc_params,
        in_specs=[
            pl.BlockSpec((HID, SBLK), lambda j: (0, j)),
            pl.BlockSpec((HID, SBLK), lambda j: (0, j)),
            pl.BlockSpec((1, SBLK), lambda j: (0, j)),
            pl.BlockSpec((INF, SBLK), lambda j: (0, j)),
            pl.BlockSpec((2 * TPB, SBLK), lambda j, fi=fi: (fi, j)),
            pl.BlockSpec((1, TPB, SBLK), lambda j: (0, 0, j)),
            pl.BlockSpec((1, SBLK), lambda j: (0, j)),
            fullw(w0t.shape), fullw(w1t.shape), fullw(bc2.shape),
            fullw(wfull.shape), fullw(sel2.shape),
            fullw(Wo.shape), fullw(bo2.shape),
        ],
        out_specs=[
            pl.BlockSpec((HID, SBLK), lambda j: (0, j)),
            pl.BlockSpec((HID, SBLK), lambda j: (0, j)),
            pl.BlockSpec((1, SBLK), lambda j: (0, j)),
            pl.BlockSpec((1, SBLK), lambda j: (0, j)),
        ],
        out_shape=[
            jax.ShapeDtypeStruct((HID, N), jnp.float32),
            jax.ShapeDtypeStruct((HID, N), jnp.float32),
            jax.ShapeDtypeStruct((1, N), jnp.float32),
            jax.ShapeDtypeStruct((1, N), jnp.float32),
        ],
    )
    h, c, xn, axn = step(h, c, xn, feat_i, zp, sraw, dinv,
                         w0t, w1t, bc2, wfull, sel2, Wo, bo2)
    preds.append(xn)
  out = jnp.concatenate(preds, axis=0).reshape(PRED, B, C).transpose(1, 0, 2)
  return out


# speedup vs baseline: 1.1173x; 1.0424x over previous
import functools

import jax
import jax.numpy as jnp
from jax import lax
from jax.experimental import pallas as pl
from jax.experimental.pallas import tpu as pltpu
from jax.experimental.pallas import tpu_sc as plsc

NC = 2
NS = 16
NW = NC * NS
_SC_PARAMS = pltpu.CompilerParams(needs_layout_passes=False)


def _tile_ids(TPB):
  cid = lax.axis_index("c")
  sid = lax.axis_index("s")
  wid = cid * NS + sid
  return wid, wid // TPB, wid % TPB


def _make_edges_deg(B, C, EPB):
  TPB = NW // B
  EPT = EPB // TPB
  ITERS = EPT // 16
  E2 = B * EPB
  SH = max((C - 1).bit_length(), 1)
  mesh = plsc.VectorSubcoreMesh(core_axis_name="c", subcore_axis_name="s")

  @functools.partial(
      pl.kernel,
      out_type=[
          jax.ShapeDtypeStruct((E2,), jnp.int32),
          jax.ShapeDtypeStruct((TPB * B * C,), jnp.float32),
      ],
      mesh=mesh,
      compiler_params=_SC_PARAMS,
      scratch_types=[
          pltpu.VMEM((EPT,), jnp.int32),
          pltpu.VMEM((EPT,), jnp.int32),
          pltpu.VMEM((C,), jnp.float32),
      ],
  )
  def edges_deg(ei_hbm, eloc_hbm, deg_hbm, src_v, dst_v, acc_v):
    wid, b, q = _tile_ids(TPB)
    e0 = wid * EPT
    pltpu.sync_copy(ei_hbm.at[pl.ds(e0, EPT)], src_v)
    pltpu.sync_copy(ei_hbm.at[pl.ds(E2 + e0, EPT)], dst_v)
    offv = jnp.full((16,), b * C, jnp.int32)
    zv = jnp.zeros((16,), jnp.float32)
    ones = jnp.ones((16,), jnp.float32)

    @plsc.parallel_loop(0, C // 16, 1, unroll=4)
    def _(i):
      acc_v[pl.ds(i * 16, 16)] = zv

    @plsc.parallel_loop(0, ITERS, 1, unroll=4)
    def _(i):
      dv = dst_v[pl.ds(i * 16, 16)] - offv
      sv = src_v[pl.ds(i * 16, 16)] - offv
      plsc.addupdate_scatter(acc_v, [dv], ones)
      src_v[pl.ds(i * 16, 16)] = jnp.left_shift(dv, SH) + sv

    pltpu.sync_copy(src_v, eloc_hbm.at[pl.ds(e0, EPT)])
    pltpu.sync_copy(acc_v, deg_hbm.at[pl.ds(q * (B * C) + b * C, C)])

  return edges_deg


def _make_spmv(Ttot, t0, T, B, C, EPB, CG):
  TPB = NW // B
  EPT = EPB // TPB
  ITERS = EPT // 16
  SH = max((C - 1).bit_length(), 1)
  MASK = (1 << SH) - 1
  mesh = plsc.VectorSubcoreMesh(core_axis_name="c", subcore_axis_name="s")

  @functools.partial(
      pl.kernel,
      out_type=jax.ShapeDtypeStruct((T * TPB * B * C,), jnp.float32),
      mesh=mesh,
      compiler_params=_SC_PARAMS,
      scratch_types=(
          [pltpu.VMEM((EPT,), jnp.int32)]
          + [pltpu.VMEM((C,), jnp.float32) for _ in range(2 * CG)]
      ),
  )
  def spmv(y_hbm, eloc_hbm, out_hbm, idx_v, *tv):
    tabs, accs = tv[:CG], tv[CG:]
    wid, b, q = _tile_ids(TPB)
    e0 = wid * EPT
    pltpu.sync_copy(eloc_hbm.at[pl.ds(e0, EPT)], idx_v)
    zv = jnp.zeros((16,), jnp.float32)
    maskv = jnp.full((16,), MASK, jnp.int32)

    for g0 in range(0, T, CG):
      for gi in range(CG):
        pltpu.sync_copy(
            y_hbm.at[pl.ds((t0 + g0 + gi) * (B * C) + b * C, C)], tabs[gi])

      @plsc.parallel_loop(0, C // 16, 1, unroll=4)
      def _(i):
        for gi in range(CG):
          accs[gi][pl.ds(i * 16, 16)] = zv

      @plsc.parallel_loop(0, ITERS, 1, unroll=4)
      def _(i):
        ev = idx_v[pl.ds(i * 16, 16)]
        sv = jnp.bitwise_and(ev, maskv)
        dv = jnp.right_shift(ev, SH)
        for gi in range(CG):
          vals = plsc.load_gather(tabs[gi], [sv])
          plsc.addupdate_scatter(accs[gi], [dv], vals)

      for gi in range(CG):
        pltpu.sync_copy(
            accs[gi],
            out_hbm.at[pl.ds(((g0 + gi) * TPB + q) * (B * C) + b * C, C)])

  return spmv


def _pick_blk(n):
  for blk in (3200, 640, 1280, 512, 256, 128):
    if n % blk == 0:
      return blk
  return n


def _sig(x):
  return 0.5 * jnp.tanh(0.5 * x) + 0.5


def _lstm(gates, c, HID):
  ig = _sig(gates[0 * HID:1 * HID])
  fg = _sig(gates[1 * HID:2 * HID])
  gg = jnp.tanh(gates[2 * HID:3 * HID])
  og = _sig(gates[3 * HID:4 * HID])
  c_new = c * fg + ig * gg
  h_new = og * jnp.tanh(c_new)
  return h_new, c_new


def _prep_kernel(deg4_ref, pm25_ref, feat_ref, w1blk_ref, pblk_ref,
                 dinv_ref, ysc_ref):
  deg = jnp.sum(deg4_ref[...], axis=0)
  dinv = jnp.where(deg > 0, lax.rsqrt(jnp.maximum(deg, 1e-12)), 0.0)
  dinv_ref[...] = dinv[None, :]
  q_all = (jnp.dot(w1blk_ref[...], feat_ref[...],
                   preferred_element_type=jnp.float32)
           + jnp.dot(pblk_ref[...], pm25_ref[...],
                     preferred_element_type=jnp.float32))
  ysc_ref[...] = dinv[None, :] * q_all


def _xg(a, f9, zsum, dinv, w0t, bc, extra):
  pre = jnp.dot(w0t[:, 1:], f9, preferred_element_type=jnp.float32)
  x = w0t[:, 0:1] * a[None, :] + pre + bc - dinv[None, :] * zsum
  if extra is not None:
    x = x + extra
  return _sig(x)


def _hist_kernel(pm25_ref, feat_ref, z_ref, dinv_ref, w0blk_ref, p0blk_ref,
                 bcrep_ref, wfull_ref, sel_ref, wo_ref, bo_ref,
                 h_ref, c_ref, xn_ref, axn_ref, *, HIST, HID, BLK, INF):
  dinv = dinv_ref[0]
  wfull = wfull_ref[...]
  zs = jnp.dot(sel_ref[...], z_ref[...],
               preferred_element_type=jnp.float32)
  xg_all = _sig(jnp.dot(w0blk_ref[...], feat_ref[...],
                        preferred_element_type=jnp.float32)
                + jnp.dot(p0blk_ref[...], pm25_ref[...],
                          preferred_element_type=jnp.float32)
                + bcrep_ref[...] - dinv[None, :] * zs)
  ones_row = jnp.ones((1, BLK), jnp.float32)
  h = jnp.zeros((HID, BLK), jnp.float32)
  c = jnp.zeros((HID, BLK), jnp.float32)
  for s in range(HIST):
    a = pm25_ref[s]
    f9 = feat_ref[INF * s:INF * (s + 1)]
    xx = jnp.concatenate(
        [a[None, :], f9, xg_all[2 * s:2 * s + 2], h, ones_row], axis=0)
    gates = jnp.dot(wfull, xx, preferred_element_type=jnp.float32)
    h, c = _lstm(gates, c, HID)
  wo = wo_ref[...]
  xn = jnp.dot(wo, h, preferred_element_type=jnp.float32) + bo_ref[0, 0]
  h_ref[...] = h
  c_ref[...] = c
  xn_ref[...] = xn
  axn_ref[...] = dinv[None, :] * xn


def _step_kernel(h_in_ref, c_in_ref, xn_in_ref, feat_ref, z_ref, s_ref,
                 dinv_ref, w0t_ref, w1t_ref, bc_ref, wfull_ref, sel2_ref,
                 wo_ref, bo_ref, h_ref, c_ref, xn_ref, axn_ref,
                 *, HID, BLK):
  dinv = dinv_ref[0]
  a = xn_in_ref[0]
  f9 = feat_ref[...]
  zsum = jnp.dot(sel2_ref[...], z_ref[...],
                 preferred_element_type=jnp.float32)
  sd = dinv * jnp.sum(s_ref[0], axis=0)
  extra = -w1t_ref[...][:, 0:1] * sd[None, :]
  xg = _xg(a, f9, zsum, dinv, w0t_ref[...], bc_ref[...], extra)
  ones_row = jnp.ones((1, BLK), jnp.float32)
  xx = jnp.concatenate([a[None, :], f9, xg, h_in_ref[...], ones_row], axis=0)
  gates = jnp.dot(wfull_ref[...], xx, preferred_element_type=jnp.float32)
  h, c = _lstm(gates, c_in_ref[...], HID)
  xn = jnp.dot(wo_ref[...], h, preferred_element_type=jnp.float32) + bo_ref[0, 0]
  h_ref[...] = h
  c_ref[...] = c
  xn_ref[...] = xn
  axn_ref[...] = dinv[None, :] * xn


def kernel(pm25_hist, feature, edge_index, W0, W1, bC, Wx, bx, Wh, bh, Wo, bo):
  B, HIST, C = pm25_hist.shape
  PRED = feature.shape[1] - HIST
  T = HIST + PRED
  N = B * C
  INF = feature.shape[3]
  HID = Wh.shape[1]
  E2 = edge_index.shape[1]
  EPB = E2 // B
  TPB = NW // B

  eiflat = edge_index.astype(jnp.int32).reshape(-1)

  eloc, deg4 = _make_edges_deg(B, C, EPB)(eiflat)
  deg4 = deg4.reshape(TPB, N)

  pm25T = pm25_hist.transpose(1, 0, 2).reshape(HIST, N)
  featT = feature.transpose(1, 3, 0, 2).reshape(T * INF, N)
  GCN = W0.shape[1]
  w0t = W0.T
  w1t = W1.T
  bc2 = bC.reshape(-1, 1)
  bxh = (bx + bh).reshape(-1, 1)
  bo2 = bo.reshape(1, 1)
  wfull = jnp.concatenate([Wx, Wh, bxh], axis=1)
  w1blk = jnp.kron(jnp.eye(T, dtype=jnp.float32), W1[1:].T)
  pblk = jnp.kron(jnp.eye(T, dtype=jnp.float32),
                  W1[0:1].T)[:, :HIST]
  sel = jnp.kron(jnp.eye(2 * HIST, dtype=jnp.float32),
                 jnp.ones((1, TPB), jnp.float32))
  sel2 = jnp.kron(jnp.eye(2, dtype=jnp.float32),
                  jnp.ones((1, TPB), jnp.float32))
  w0blk = jnp.kron(jnp.eye(HIST, dtype=jnp.float32), W0[1:].T)
  p0blk = jnp.kron(jnp.eye(HIST, dtype=jnp.float32), W0[0:1].T)
  bcrep = jnp.tile(bc2, (HIST, 1))

  BLK = _pick_blk(N)
  grid = (N // BLK,)
  tc_params = pltpu.CompilerParams(dimension_semantics=("parallel",))
  fullw = lambda shape: pl.BlockSpec(shape, lambda j: (0,) * len(shape))

  dinv, ysc = pl.pallas_call(
      _prep_kernel,
      grid=grid,
      compiler_params=tc_params,
      in_specs=[
          pl.BlockSpec((TPB, BLK), lambda j: (0, j)),
          pl.BlockSpec((HIST, BLK), lambda j: (0, j)),
          pl.BlockSpec((T * INF, BLK), lambda j: (0, j)),
          fullw(w1blk.shape), fullw(pblk.shape),
      ],
      out_specs=[
          pl.BlockSpec((1, BLK), lambda j: (0, j)),
          pl.BlockSpec((2 * T, BLK), lambda j: (0, j)),
      ],
      out_shape=[
          jax.ShapeDtypeStruct((1, N), jnp.float32),
          jax.ShapeDtypeStruct((2 * T, N), jnp.float32),
      ],
  )(deg4, pm25T, featT, w1blk, pblk)

  yflat = ysc.reshape(-1)
  zh = _make_spmv(2 * T, 0, 2 * HIST, B, C, EPB, 4)(yflat, eloc)
  zh = zh.reshape(2 * HIST * TPB, N)
  zp = _make_spmv(2 * T, 2 * HIST, 2 * PRED, B, C, EPB, 4)(yflat, eloc)
  zp = zp.reshape(2 * PRED * TPB, N)

  h, c, xn, axn = pl.pallas_call(
      functools.partial(_hist_kernel, HIST=HIST, HID=HID, BLK=BLK, INF=INF),
      grid=grid,
      compiler_params=tc_params,
      in_specs=[
          pl.BlockSpec((HIST, BLK), lambda j: (0, j)),
          pl.BlockSpec((HIST * INF, BLK), lambda j: (0, j)),
          pl.BlockSpec((2 * HIST * TPB, BLK), lambda j: (0, j)),
          pl.BlockSpec((1, BLK), lambda j: (0, j)),
          fullw(w0blk.shape), fullw(p0blk.shape), fullw(bcrep.shape),
          fullw(wfull.shape), fullw(sel.shape), fullw(Wo.shape),
          fullw(bo2.shape),
      ],
      out_specs=[
          pl.BlockSpec((HID, BLK), lambda j: (0, j)),
          pl.BlockSpec((HID, BLK), lambda j: (0, j)),
          pl.BlockSpec((1, BLK), lambda j: (0, j)),
          pl.BlockSpec((1, BLK), lambda j: (0, j)),
      ],
      out_shape=[
          jax.ShapeDtypeStruct((HID, N), jnp.float32),
          jax.ShapeDtypeStruct((HID, N), jnp.float32),
          jax.ShapeDtypeStruct((1, N), jnp.float32),
          jax.ShapeDtypeStruct((1, N), jnp.float32),
      ],
  )(pm25T, featT, zh, dinv, w0blk, p0blk, bcrep, wfull, sel, Wo, bo2)

  spmv1 = _make_spmv(1, 0, 1, B, C, EPB, 1)
  SBLK = 16000 if N % 16000 == 0 else BLK
  sgrid = (N // SBLK,)

  preds = []
  for i in range(PRED):
    sraw = spmv1(axn.reshape(-1), eloc).reshape(1, TPB, N)
    fi = i
    feat_i = lax.slice_in_dim(featT, (HIST + i) * INF, (HIST + i + 1) * INF,
                              axis=0)

    step = pl.pallas_call(
        functools.partial(_step_kernel, HID=HID, BLK=SBLK),
        grid=sgrid,
        compiler_params=tc_params,
        in_specs=[
            pl.BlockSpec((HID, SBLK), lambda j: (0, j)),
            pl.BlockSpec((HID, SBLK), lambda j: (0, j)),
            pl.BlockSpec((1, SBLK), lambda j: (0, j)),
            pl.BlockSpec((INF, SBLK), lambda j: (0, j)),
            pl.BlockSpec((2 * TPB, SBLK), lambda j, fi=fi: (fi, j)),
            pl.BlockSpec((1, TPB, SBLK), lambda j: (0, 0, j)),
            pl.BlockSpec((1, SBLK), lambda j: (0, j)),
            fullw(w0t.shape), fullw(w1t.shape), fullw(bc2.shape),
            fullw(wfull.shape), fullw(sel2.shape),
            fullw(Wo.shape), fullw(bo2.shape),
        ],
        out_specs=[
            pl.BlockSpec((HID, SBLK), lambda j: (0, j)),
            pl.BlockSpec((HID, SBLK), lambda j: (0, j)),
            pl.BlockSpec((1, SBLK), lambda j: (0, j)),
            pl.BlockSpec((1, SBLK), lambda j: (0, j)),
        ],
        out_shape=[
            jax.ShapeDtypeStruct((HID, N), jnp.float32),
            jax.ShapeDtypeStruct((HID, N), jnp.float32),
            jax.ShapeDtypeStruct((1, N), jnp.float32),
            jax.ShapeDtypeStruct((1, N), jnp.float32),
        ],
    )
    h, c, xn, axn = step(h, c, xn, feat_i, zp, sraw, dinv,
                         w0t, w1t, bc2, wfull, sel2, Wo, bo2)
    preds.append(xn)
  out = jnp.concatenate(preds, axis=0).reshape(PRED, B, C).transpose(1, 0, 2)
  return out


# speedup vs baseline: 1.1207x; 1.0031x over previous
import functools

import jax
import jax.numpy as jnp
from jax import lax
from jax.experimental import pallas as pl
from jax.experimental.pallas import tpu as pltpu
from jax.experimental.pallas import tpu_sc as plsc

NC = 2
NS = 16
NW = NC * NS
_SC_PARAMS = pltpu.CompilerParams(needs_layout_passes=False)


def _tile_ids(TPB):
  cid = lax.axis_index("c")
  sid = lax.axis_index("s")
  wid = cid * NS + sid
  return wid, wid // TPB, wid % TPB


def _make_edges_deg(B, C, EPB):
  TPB = NW // B
  EPT = EPB // TPB
  ITERS = EPT // 16
  E2 = B * EPB
  SH = max((C - 1).bit_length(), 1)
  mesh = plsc.VectorSubcoreMesh(core_axis_name="c", subcore_axis_name="s")

  @functools.partial(
      pl.kernel,
      out_type=[
          jax.ShapeDtypeStruct((E2,), jnp.int32),
          jax.ShapeDtypeStruct((TPB * B * C,), jnp.float32),
      ],
      mesh=mesh,
      compiler_params=_SC_PARAMS,
      scratch_types=[
          pltpu.VMEM((EPT,), jnp.int32),
          pltpu.VMEM((EPT,), jnp.int32),
          pltpu.VMEM((C,), jnp.float32),
      ],
  )
  def edges_deg(ei_hbm, eloc_hbm, deg_hbm, src_v, dst_v, acc_v):
    wid, b, q = _tile_ids(TPB)
    e0 = wid * EPT
    pltpu.sync_copy(ei_hbm.at[pl.ds(e0, EPT)], src_v)
    pltpu.sync_copy(ei_hbm.at[pl.ds(E2 + e0, EPT)], dst_v)
    offv = jnp.full((16,), b * C, jnp.int32)
    zv = jnp.zeros((16,), jnp.float32)
    ones = jnp.ones((16,), jnp.float32)

    @plsc.parallel_loop(0, C // 16, 1, unroll=4)
    def _(i):
      acc_v[pl.ds(i * 16, 16)] = zv

    @plsc.parallel_loop(0, ITERS, 1, unroll=4)
    def _(i):
      dv = dst_v[pl.ds(i * 16, 16)] - offv
      sv = src_v[pl.ds(i * 16, 16)] - offv
      plsc.addupdate_scatter(acc_v, [dv], ones)
      src_v[pl.ds(i * 16, 16)] = jnp.left_shift(dv, SH) + sv

    pltpu.sync_copy(src_v, eloc_hbm.at[pl.ds(e0, EPT)])
    pltpu.sync_copy(acc_v, deg_hbm.at[pl.ds(q * (B * C) + b * C, C)])

  return edges_deg


def _make_spmv(Ttot, t0, T, B, C, EPB, CG):
  TPB = NW // B
  EPT = EPB // TPB
  ITERS = EPT // 16
  SH = max((C - 1).bit_length(), 1)
  MASK = (1 << SH) - 1
  mesh = plsc.VectorSubcoreMesh(core_axis_name="c", subcore_axis_name="s")

  @functools.partial(
      pl.kernel,
      out_type=jax.ShapeDtypeStruct((T * TPB * B * C,), jnp.float32),
      mesh=mesh,
      compiler_params=_SC_PARAMS,
      scratch_types=(
          [pltpu.VMEM((EPT,), jnp.int32)]
          + [pltpu.VMEM((C,), jnp.float32) for _ in range(2 * CG)]
      ),
  )
  def spmv(y_hbm, eloc_hbm, out_hbm, idx_v, *tv):
    tabs, accs = tv[:CG], tv[CG:]
    wid, b, q = _tile_ids(TPB)
    e0 = wid * EPT
    pltpu.sync_copy(eloc_hbm.at[pl.ds(e0, EPT)], idx_v)
    zv = jnp.zeros((16,), jnp.float32)
    maskv = jnp.full((16,), MASK, jnp.int32)

    for g0 in range(0, T, CG):
      for gi in range(CG):
        pltpu.sync_copy(
            y_hbm.at[pl.ds((t0 + g0 + gi) * (B * C) + b * C, C)], tabs[gi])

      @plsc.parallel_loop(0, C // 16, 1, unroll=4)
      def _(i):
        for gi in range(CG):
          accs[gi][pl.ds(i * 16, 16)] = zv

      @plsc.parallel_loop(0, ITERS, 1, unroll=2)
      def _(i):
        ev = idx_v[pl.ds(i * 16, 16)]
        sv = jnp.bitwise_and(ev, maskv)
        dv = jnp.right_shift(ev, SH)
        for gi in range(CG):
          vals = plsc.load_gather(tabs[gi], [sv])
          plsc.addupdate_scatter(accs[gi], [dv], vals)

      for gi in range(CG):
        pltpu.sync_copy(
            accs[gi],
            out_hbm.at[pl.ds(((g0 + gi) * TPB + q) * (B * C) + b * C, C)])

  return spmv


def _pick_blk(n):
  for blk in (3200, 640, 1280, 512, 256, 128):
    if n % blk == 0:
      return blk
  return n


def _sig(x):
  return 0.5 * jnp.tanh(0.5 * x) + 0.5


def _lstm(gates, c, HID):
  ig = _sig(gates[0 * HID:1 * HID])
  fg = _sig(gates[1 * HID:2 * HID])
  gg = jnp.tanh(gates[2 * HID:3 * HID])
  og = _sig(gates[3 * HID:4 * HID])
  c_new = c * fg + ig * gg
  h_new = og * jnp.tanh(c_new)
  return h_new, c_new


def _prep_kernel(deg4_ref, pm25_ref, feat_ref, w1blk_ref, pblk_ref,
                 dinv_ref, ysc_ref):
  deg = jnp.sum(deg4_ref[...], axis=0)
  dinv = jnp.where(deg > 0, lax.rsqrt(jnp.maximum(deg, 1e-12)), 0.0)
  dinv_ref[...] = dinv[None, :]
  q_all = (jnp.dot(w1blk_ref[...], feat_ref[...],
                   preferred_element_type=jnp.float32)
           + jnp.dot(pblk_ref[...], pm25_ref[...],
                     preferred_element_type=jnp.float32))
  ysc_ref[...] = dinv[None, :] * q_all


def _xg(a, f9, zsum, dinv, w0t, bc, extra):
  pre = jnp.dot(w0t[:, 1:], f9, preferred_element_type=jnp.float32)
  x = w0t[:, 0:1] * a[None, :] + pre + bc - dinv[None, :] * zsum
  if extra is not None:
    x = x + extra
  return _sig(x)


def _hist_kernel(pm25_ref, feat_ref, z_ref, dinv_ref, w0blk_ref, p0blk_ref,
                 bcrep_ref, wfull_ref, sel_ref, wo_ref, bo_ref,
                 h_ref, c_ref, xn_ref, axn_ref, *, HIST, HID, BLK, INF):
  dinv = dinv_ref[0]
  wfull = wfull_ref[...]
  zs = jnp.dot(sel_ref[...], z_ref[...],
               preferred_element_type=jnp.float32)
  xg_all = _sig(jnp.dot(w0blk_ref[...], feat_ref[...],
                        preferred_element_type=jnp.float32)
                + jnp.dot(p0blk_ref[...], pm25_ref[...],
                          preferred_element_type=jnp.float32)
                + bcrep_ref[...] - dinv[None, :] * zs)
  ones_row = jnp.ones((1, BLK), jnp.float32)
  h = jnp.zeros((HID, BLK), jnp.float32)
  c = jnp.zeros((HID, BLK), jnp.float32)
  for s in range(HIST):
    a = pm25_ref[s]
    f9 = feat_ref[INF * s:INF * (s + 1)]
    xx = jnp.concatenate(
        [a[None, :], f9, xg_all[2 * s:2 * s + 2], h, ones_row], axis=0)
    gates = jnp.dot(wfull, xx, preferred_element_type=jnp.float32)
    h, c = _lstm(gates, c, HID)
  wo = wo_ref[...]
  xn = jnp.dot(wo, h, preferred_element_type=jnp.float32) + bo_ref[0, 0]
  h_ref[...] = h
  c_ref[...] = c
  xn_ref[...] = xn
  axn_ref[...] = dinv[None, :] * xn


def _step_kernel(h_in_ref, c_in_ref, xn_in_ref, feat_ref, z_ref, s_ref,
                 dinv_ref, w0t_ref, w1t_ref, bc_ref, wfull_ref, sel2_ref,
                 wo_ref, bo_ref, h_ref, c_ref, xn_ref, axn_ref,
                 *, HID, BLK):
  dinv = dinv_ref[0]
  a = xn_in_ref[0]
  f9 = feat_ref[...]
  zsum = jnp.dot(sel2_ref[...], z_ref[...],
                 preferred_element_type=jnp.float32)
  sd = dinv * jnp.sum(s_ref[0], axis=0)
  extra = -w1t_ref[...][:, 0:1] * sd[None, :]
  xg = _xg(a, f9, zsum, dinv, w0t_ref[...], bc_ref[...], extra)
  ones_row = jnp.ones((1, BLK), jnp.float32)
  xx = jnp.concatenate([a[None, :], f9, xg, h_in_ref[...], ones_row], axis=0)
  gates = jnp.dot(wfull_ref[...], xx, preferred_element_type=jnp.float32)
  h, c = _lstm(gates, c_in_ref[...], HID)
  xn = jnp.dot(wo_ref[...], h, preferred_element_type=jnp.float32) + bo_ref[0, 0]
  h_ref[...] = h
  c_ref[...] = c
  xn_ref[...] = xn
  axn_ref[...] = dinv[None, :] * xn


def kernel(pm25_hist, feature, edge_index, W0, W1, bC, Wx, bx, Wh, bh, Wo, bo):
  B, HIST, C = pm25_hist.shape
  PRED = feature.shape[1] - HIST
  T = HIST + PRED
  N = B * C
  INF = feature.shape[3]
  HID = Wh.shape[1]
  E2 = edge_index.shape[1]
  EPB = E2 // B
  TPB = NW // B

  eiflat = edge_index.astype(jnp.int32).reshape(-1)

  eloc, deg4 = _make_edges_deg(B, C, EPB)(eiflat)
  deg4 = deg4.reshape(TPB, N)

  pm25T = pm25_hist.transpose(1, 0, 2).reshape(HIST, N)
  featT = feature.transpose(1, 3, 0, 2).reshape(T * INF, N)
  GCN = W0.shape[1]
  w0t = W0.T
  w1t = W1.T
  bc2 = bC.reshape(-1, 1)
  bxh = (bx + bh).reshape(-1, 1)
  bo2 = bo.reshape(1, 1)
  wfull = jnp.concatenate([Wx, Wh, bxh], axis=1)
  w1blk = jnp.kron(jnp.eye(T, dtype=jnp.float32), W1[1:].T)
  pblk = jnp.kron(jnp.eye(T, dtype=jnp.float32),
                  W1[0:1].T)[:, :HIST]
  sel = jnp.kron(jnp.eye(2 * HIST, dtype=jnp.float32),
                 jnp.ones((1, TPB), jnp.float32))
  sel2 = jnp.kron(jnp.eye(2, dtype=jnp.float32),
                  jnp.ones((1, TPB), jnp.float32))
  w0blk = jnp.kron(jnp.eye(HIST, dtype=jnp.float32), W0[1:].T)
  p0blk = jnp.kron(jnp.eye(HIST, dtype=jnp.float32), W0[0:1].T)
  bcrep = jnp.tile(bc2, (HIST, 1))

  BLK = _pick_blk(N)
  grid = (N // BLK,)
  tc_params = pltpu.CompilerParams(dimension_semantics=("parallel",))
  fullw = lambda shape: pl.BlockSpec(shape, lambda j: (0,) * len(shape))

  dinv, ysc = pl.pallas_call(
      _prep_kernel,
      grid=grid,
      compiler_params=tc_params,
      in_specs=[
          pl.BlockSpec((TPB, BLK), lambda j: (0, j)),
          pl.BlockSpec((HIST, BLK), lambda j: (0, j)),
          pl.BlockSpec((T * INF, BLK), lambda j: (0, j)),
          fullw(w1blk.shape), fullw(pblk.shape),
      ],
      out_specs=[
          pl.BlockSpec((1, BLK), lambda j: (0, j)),
          pl.BlockSpec((2 * T, BLK), lambda j: (0, j)),
      ],
      out_shape=[
          jax.ShapeDtypeStruct((1, N), jnp.float32),
          jax.ShapeDtypeStruct((2 * T, N), jnp.float32),
      ],
  )(deg4, pm25T, featT, w1blk, pblk)

  yflat = ysc.reshape(-1)
  zh = _make_spmv(2 * T, 0, 2 * HIST, B, C, EPB, 4)(yflat, eloc)
  zh = zh.reshape(2 * HIST * TPB, N)
  zp = _make_spmv(2 * T, 2 * HIST, 2 * PRED, B, C, EPB, 4)(yflat, eloc)
  zp = zp.reshape(2 * PRED * TPB, N)

  h, c, xn, axn = pl.pallas_call(
      functools.partial(_hist_kernel, HIST=HIST, HID=HID, BLK=BLK, INF=INF),
      grid=grid,
      compiler_params=tc_params,
      in_specs=[
          pl.BlockSpec((HIST, BLK), lambda j: (0, j)),
          pl.BlockSpec((HIST * INF, BLK), lambda j: (0, j)),
          pl.BlockSpec((2 * HIST * TPB, BLK), lambda j: (0, j)),
          pl.BlockSpec((1, BLK), lambda j: (0, j)),
          fullw(w0blk.shape), fullw(p0blk.shape), fullw(bcrep.shape),
          fullw(wfull.shape), fullw(sel.shape), fullw(Wo.shape),
          fullw(bo2.shape),
      ],
      out_specs=[
          pl.BlockSpec((HID, BLK), lambda j: (0, j)),
          pl.BlockSpec((HID, BLK), lambda j: (0, j)),
          pl.BlockSpec((1, BLK), lambda j: (0, j)),
          pl.BlockSpec((1, BLK), lambda j: (0, j)),
      ],
      out_shape=[
          jax.ShapeDtypeStruct((HID, N), jnp.float32),
          jax.ShapeDtypeStruct((HID, N), jnp.float32),
          jax.ShapeDtypeStruct((1, N), jnp.float32),
          jax.ShapeDtypeStruct((1, N), jnp.float32),
      ],
  )(pm25T, featT, zh, dinv, w0blk, p0blk, bcrep, wfull, sel, Wo, bo2)

  spmv1 = _make_spmv(1, 0, 1, B, C, EPB, 1)
  SBLK = 16000 if N % 16000 == 0 else BLK
  sgrid = (N // SBLK,)

  preds = []
  for i in range(PRED):
    sraw = spmv1(axn.reshape(-1), eloc).reshape(1, TPB, N)
    fi = i
    feat_i = lax.slice_in_dim(featT, (HIST + i) * INF, (HIST + i + 1) * INF,
                              axis=0)

    step = pl.pallas_call(
        functools.partial(_step_kernel, HID=HID, BLK=SBLK),
        grid=sgrid,
        compiler_params=tc_params,
        in_specs=[
            pl.BlockSpec((HID, SBLK), lambda j: (0, j)),
            pl.BlockSpec((HID, SBLK), lambda j: (0, j)),
            pl.BlockSpec((1, SBLK), lambda j: (0, j)),
            pl.BlockSpec((INF, SBLK), lambda j: (0, j)),
            pl.BlockSpec((2 * TPB, SBLK), lambda j, fi=fi: (fi, j)),
            pl.BlockSpec((1, TPB, SBLK), lambda j: (0, 0, j)),
            pl.BlockSpec((1, SBLK), lambda j: (0, j)),
            fullw(w0t.shape), fullw(w1t.shape), fullw(bc2.shape),
            fullw(wfull.shape), fullw(sel2.shape),
            fullw(Wo.shape), fullw(bo2.shape),
        ],
        out_specs=[
            pl.BlockSpec((HID, SBLK), lambda j: (0, j)),
            pl.BlockSpec((HID, SBLK), lambda j: (0, j)),
            pl.BlockSpec((1, SBLK), lambda j: (0, j)),
            pl.BlockSpec((1, SBLK), lambda j: (0, j)),
        ],
        out_shape=[
            jax.ShapeDtypeStruct((HID, N), jnp.float32),
            jax.ShapeDtypeStruct((HID, N), jnp.float32),
            jax.ShapeDtypeStruct((1, N), jnp.float32),
            jax.ShapeDtypeStruct((1, N), jnp.float32),
        ],
    )
    h, c, xn, axn = step(h, c, xn, feat_i, zp, sraw, dinv,
                         w0t, w1t, bc2, wfull, sel2, Wo, bo2)
    preds.append(xn)
  out = jnp.concatenate(preds, axis=0).reshape(PRED, B, C).transpose(1, 0, 2)
  return out
